# Initial kernel scaffold; baseline (speedup 1.0000x reference)
import functools
import math

import jax
import jax.numpy as jnp
from jax import lax
from jax.experimental import pallas as pl
from jax.experimental.pallas import tpu as pltpu
from jax.experimental.pallas import tpu_sc as plsc

KNN = 10
INV = 1.0 / math.sqrt(1.0 + 1e-5)
NEG = -3.0e38


def _lrelu(x):
    return jnp.where(x >= 0, x, 0.2 * x)


def _knn_proj_body(xt_ref, xc_ref, w_ref, idx_ref, at_ref, ct_ref, *, n, r, o):
    b = pl.program_id(0)
    xr = xt_ref[0]
    xc = xc_ref[0]
    g = jnp.dot(xr, xc, preferred_element_type=jnp.float32,
                precision=lax.Precision.HIGHEST)
    xx_r = jnp.sum(xr * xr, axis=1, keepdims=True)
    xx_c = jnp.sum(xc * xc, axis=0, keepdims=True)
    key = -xx_r - (-2.0 * g) - xx_c
    iota = lax.broadcasted_iota(jnp.int32, (r, n), 1)
    cols = []
    for _ in range(KNN):
        m = jnp.max(key, axis=1, keepdims=True)
        cand = jnp.where(key >= m, iota, n)
        ii = jnp.min(cand, axis=1, keepdims=True)
        cols.append(ii)
        key = jnp.where(iota == ii, NEG, key)
    idx_ref[0] = jnp.concatenate(cols, axis=1) + b * n

    p = jnp.dot(xr, w_ref[...], preferred_element_type=jnp.float32,
                precision=lax.Precision.HIGHEST)
    at_ref[0] = p[:, :o]
    ct_ref[0] = p[:, o:]


def _knn_proj(xt, xcm, wcat, o, r=256):
    b, n, c = xt.shape
    body = functools.partial(_knn_proj_body, n=n, r=r, o=o)
    return pl.pallas_call(
        body,
        grid=(b, n // r),
        in_specs=[
            pl.BlockSpec((1, r, c), lambda i, j: (i, j, 0)),
            pl.BlockSpec((1, c, n), lambda i, j: (i, 0, 0)),
            pl.BlockSpec((c, 2 * o), lambda i, j: (0, 0)),
        ],
        out_specs=[
            pl.BlockSpec((1, r, KNN), lambda i, j: (i, j, 0)),
            pl.BlockSpec((1, r, o), lambda i, j: (i, j, 0)),
            pl.BlockSpec((1, r, o), lambda i, j: (i, j, 0)),
        ],
        out_shape=[
            jax.ShapeDtypeStruct((b, n, KNN), jnp.int32),
            jax.ShapeDtypeStruct((b, n, o), jnp.float32),
            jax.ShapeDtypeStruct((b, n, o), jnp.float32),
        ],
    )(xt, xcm, wcat)


def _sc_gather_max(at, ct, idx, gv, bv):
    bn_, o = at.shape
    nc, ns = 2, 16
    nw = nc * ns
    pw = bn_ // nw
    ch = 32
    nch = pw // ch
    mesh = plsc.VectorSubcoreMesh(core_axis_name="c", subcore_axis_name="s")

    @functools.partial(
        pl.kernel,
        out_type=jax.ShapeDtypeStruct((bn_, o), jnp.float32),
        mesh=mesh,
        scratch_types=[
            pltpu.VMEM((ch * KNN,), jnp.int32),
            pltpu.VMEM((ch * KNN, o), jnp.float32),
            pltpu.VMEM((ch, o), jnp.float32),
            pltpu.VMEM((ch, o), jnp.float32),
            pltpu.VMEM((o,), jnp.float32),
            pltpu.VMEM((o,), jnp.float32),
            pltpu.SemaphoreType.DMA,
        ],
    )
    def k(at_hbm, ct_hbm, idx_hbm, g_hbm, b_hbm, out_hbm,
          idx_v, rows_v, ct_v, out_v, g_v, b_v, sem):
        wid = lax.axis_index("s") * nc + lax.axis_index("c")
        pltpu.sync_copy(g_hbm, g_v)
        pltpu.sync_copy(b_hbm, b_v)

        def chunk_body(cix, _):
            base = wid * pw + cix * ch
            pltpu.sync_copy(idx_hbm.at[pl.ds(base * KNN, ch * KNN)], idx_v)
            pltpu.async_copy(at_hbm.at[idx_v], rows_v, sem).wait()
            pltpu.sync_copy(ct_hbm.at[pl.ds(base, ch)], ct_v)

            def pt_body(p, _):
                r0 = p * KNN
                for oc in range(o // 16):
                    sl = pl.ds(oc * 16, 16)
                    mx = rows_v[r0, sl]
                    mn = mx
                    for kk in range(1, KNN):
                        v = rows_v[r0 + kk, sl]
                        mx = jnp.maximum(mx, v)
                        mn = jnp.minimum(mn, v)
                    s = g_v[sl] * INV
                    z = jnp.where(s >= 0.0, mx, mn) + ct_v[p, sl]
                    y = s * z + b_v[sl]
                    out_v[p, sl] = jnp.where(y >= 0.0, y, 0.2 * y)
                return 0

            lax.fori_loop(0, ch, pt_body, 0)
            pltpu.sync_copy(out_v, out_hbm.at[pl.ds(base, ch)])
            return 0

        lax.fori_loop(0, nch, chunk_body, 0)

    return k(at, ct, idx, gv, bv)


def _edge_conv(xt, xcm, wcat, gv, bv, o):
    b, n, _ = xt.shape
    idx, at, ct = _knn_proj(xt, xcm, wcat, o)
    out = _sc_gather_max(at.reshape(b * n, o), ct.reshape(b * n, o),
                         idx.reshape(-1), gv, bv)
    xt2 = out.reshape(b, n, o)
    return xt2, jnp.transpose(xt2, (0, 2, 1))


def _head_body(x1_ref, x2_ref, x3_ref, x4_ref, w5t_ref, a5_ref, b5_ref,
               fwt_ref, fb_ref, oga_ref, oba_ref, swt_ref, sb_ref,
               f0_ref, f1_ref, f2_ref, f3_ref, sem_ref):
    xc = jnp.concatenate(
        [x1_ref[0], x2_ref[0], x3_ref[0], x4_ref[0]], axis=1)
    h = jnp.dot(xc, w5t_ref[...], preferred_element_type=jnp.float32,
                precision=lax.Precision.HIGHEST)
    x5 = _lrelu(a5_ref[...] * h + b5_ref[...])
    f = jnp.dot(x5, fwt_ref[...], preferred_element_type=jnp.float32,
                precision=lax.Precision.HIGHEST) + fb_ref[...]
    fb_all = oga_ref[...] * f + oba_ref[...]
    f0_ref[0] = fb_all[:, 0:256]
    f1_ref[0] = fb_all[:, 256:512]
    f2_ref[0] = fb_all[:, 512:768]
    f3_ref[0] = fb_all[:, 768:896]
    sem_ref[0] = jnp.dot(fb_all[:, 768:896], swt_ref[...],
                         preferred_element_type=jnp.float32,
                         precision=lax.Precision.HIGHEST) + sb_ref[...]


def _head(x1, x2, x3, x4, w5t, a5, b5, fwt, fb, oga, oba, swt, sb, r=512):
    b, n, _ = x1.shape

    def full(shp):
        return pl.BlockSpec(shp, lambda i, j: tuple(0 for _ in shp))

    return pl.pallas_call(
        _head_body,
        grid=(b, n // r),
        in_specs=[
            pl.BlockSpec((1, r, 64), lambda i, j: (i, j, 0)),
            pl.BlockSpec((1, r, 64), lambda i, j: (i, j, 0)),
            pl.BlockSpec((1, r, 128), lambda i, j: (i, j, 0)),
            pl.BlockSpec((1, r, 256), lambda i, j: (i, j, 0)),
            full((512, 512)), full((1, 512)), full((1, 512)),
            full((512, 896)), full((1, 896)), full((1, 896)), full((1, 896)),
            full((128, 20)), full((1, 20)),
        ],
        out_specs=[
            pl.BlockSpec((1, r, 256), lambda i, j: (i, j, 0)),
            pl.BlockSpec((1, r, 256), lambda i, j: (i, j, 0)),
            pl.BlockSpec((1, r, 256), lambda i, j: (i, j, 0)),
            pl.BlockSpec((1, r, 128), lambda i, j: (i, j, 0)),
            pl.BlockSpec((1, r, 20), lambda i, j: (i, j, 0)),
        ],
        out_shape=[
            jax.ShapeDtypeStruct((b, n, 256), jnp.float32),
            jax.ShapeDtypeStruct((b, n, 256), jnp.float32),
            jax.ShapeDtypeStruct((b, n, 256), jnp.float32),
            jax.ShapeDtypeStruct((b, n, 128), jnp.float32),
            jax.ShapeDtypeStruct((b, n, 20), jnp.float32),
        ],
    )(x1, x2, x3, x4, w5t, a5, b5, fwt, fb, oga, oba, swt, sb)


def _wcat(w, c):
    wd, wc = w[:, :c], w[:, c:]
    return jnp.concatenate([wd, wc - wd], axis=0).T


def kernel(x, W1, g1, b1, W2, g2, b2, W3, g3, b3, W4, g4, b4, W5, g5, b5,
           Fw0, Fb0, og0, ob0, Fw1, Fb1, og1, ob1, Fw2, Fb2, og2, ob2,
           Fw3, Fb3, og3, ob3, SW, Sb):
    b, _, n = x.shape
    xcm = jnp.concatenate([x, jnp.zeros((b, 5, n), jnp.float32)], axis=1)
    xt = jnp.transpose(xcm, (0, 2, 1))
    w1p = jnp.concatenate([W1[:, :3], jnp.zeros((64, 5), jnp.float32),
                           W1[:, 3:] - W1[:, :3],
                           jnp.zeros((64, 5), jnp.float32)], axis=1).T

    x1t, x1c = _edge_conv(xt, xcm, w1p, g1, b1, 64)
    x2t, x2c = _edge_conv(x1t, x1c, _wcat(W2, 64), g2, b2, 64)
    x3t, x3c = _edge_conv(x2t, x2c, _wcat(W3, 64), g3, b3, 128)
    x4t, _ = _edge_conv(x3t, x3c, _wcat(W4, 128), g4, b4, 256)

    row = lambda v: v.reshape(1, -1)
    fwt = jnp.concatenate([Fw0, Fw1, Fw2, Fw3], axis=0).T
    fb = row(jnp.concatenate([Fb0, Fb1, Fb2, Fb3]))
    oga = row(jnp.concatenate([og0, og1, og2, og3]) * INV)
    oba = row(jnp.concatenate([ob0, ob1, ob2, ob3]))
    f0, f1, f2, f3, sem = _head(
        x1t, x2t, x3t, x4t, W5.T, row(g5 * INV), row(b5),
        fwt, fb, oga, oba, SW.T, row(Sb))
    tr = lambda v: jnp.transpose(v, (0, 2, 1))
    return (tr(f0), tr(f1), tr(f2), tr(f3), sem)


# speedup vs baseline: 18.3186x; 18.3186x over previous
import functools
import math

import jax
import jax.numpy as jnp
import numpy as np
from jax import lax
from jax.experimental import pallas as pl
from jax.experimental.pallas import tpu as pltpu
from jax.experimental.pallas import tpu_sc as plsc

KNN = 10
SQRTC = float(np.sqrt(np.float32(1.0 + 1e-5)))
NEG = -3.0e38


def _lrelu(x):
    return jnp.where(x >= 0, x, 0.2 * x)


def _bn(y, g, b):
    return (g * y) / SQRTC + b


def _bf(x):
    return x.astype(jnp.bfloat16)


def _knn_body(xt_ref, xc_ref, idx_ref, *, n, r):
    b = pl.program_id(0)
    xr = xt_ref[0]
    xc = xc_ref[0]
    g = jnp.dot(_bf(xr), _bf(xc), preferred_element_type=jnp.float32)
    xx_r = jnp.sum(xr * xr, axis=1, keepdims=True)
    xx_c = jnp.sum(xc * xc, axis=0, keepdims=True)
    key = -xx_r - (-2.0 * g) - xx_c
    iota = lax.broadcasted_iota(jnp.int32, (r, n), 1)
    cols = []
    for _ in range(KNN):
        m = jnp.max(key, axis=1, keepdims=True)
        cand = jnp.where(key >= m, iota, n)
        ii = jnp.min(cand, axis=1, keepdims=True)
        cols.append(ii)
        key = jnp.where(iota == ii, NEG, key)
    idx_ref[0] = jnp.concatenate(cols, axis=1) + b * n


def _knn(xt, xcm, r=256):
    b, n, c = xt.shape
    return pl.pallas_call(
        functools.partial(_knn_body, n=n, r=r),
        grid=(b, n // r),
        in_specs=[
            pl.BlockSpec((1, r, c), lambda i, j: (i, j, 0)),
            pl.BlockSpec((1, c, n), lambda i, j: (i, 0, 0)),
        ],
        out_specs=pl.BlockSpec((1, r, KNN), lambda i, j: (i, j, 0)),
        out_shape=jax.ShapeDtypeStruct((b, n, KNN), jnp.int32),
    )(xt, xcm)


def _sc_gather(table, idx):
    bn_, cw = table.shape
    tot = idx.shape[0]
    nc, ns = 2, 16
    nw = nc * ns
    pw = tot // nw
    ch = 256
    nch = pw // ch
    mesh = plsc.VectorSubcoreMesh(core_axis_name="c", subcore_axis_name="s")

    @functools.partial(
        pl.kernel,
        out_type=jax.ShapeDtypeStruct((tot, cw), jnp.float32),
        mesh=mesh,
        scratch_types=[
            pltpu.VMEM((ch,), jnp.int32),
            pltpu.VMEM((ch, cw), jnp.float32),
            pltpu.SemaphoreType.DMA,
        ],
    )
    def k(tab_hbm, idx_hbm, out_hbm, idx_v, rows_v, sem):
        wid = lax.axis_index("s") * nc + lax.axis_index("c")

        def chunk_body(cix, _):
            base = wid * pw + cix * ch
            pltpu.sync_copy(idx_hbm.at[pl.ds(base, ch)], idx_v)
            pltpu.async_copy(tab_hbm.at[idx_v], rows_v, sem).wait()
            pltpu.sync_copy(rows_v, out_hbm.at[pl.ds(base, ch)])
            return 0

        lax.fori_loop(0, nch, chunk_body, 0)

    return k(table, idx)


def _edge_body(xg_ref, xt_ref, w_ref, g_ref, b_ref, out_ref, *, c):
    xi = xt_ref[...]
    wq = _bf(w_ref[...])
    z = None
    for k in range(KNN):
        xj = xg_ref[k][:, :c]
        ef = jnp.concatenate([xj - xi, xi], axis=1)
        y = jnp.dot(_bf(ef), wq, preferred_element_type=jnp.float32)
        t = _lrelu(_bn(y, g_ref[...], b_ref[...]))
        z = t if z is None else jnp.maximum(z, t)
    out_ref[...] = z


def _edge(xg3, xt2, wt, gv, bv, o, r=256):
    kk, bn_, cw = xg3.shape
    c = xt2.shape[1]
    return pl.pallas_call(
        functools.partial(_edge_body, c=c),
        grid=(bn_ // r,),
        in_specs=[
            pl.BlockSpec((kk, r, cw), lambda j: (0, j, 0)),
            pl.BlockSpec((r, c), lambda j: (j, 0)),
            pl.BlockSpec(wt.shape, lambda j: (0, 0)),
            pl.BlockSpec((1, o), lambda j: (0, 0)),
            pl.BlockSpec((1, o), lambda j: (0, 0)),
        ],
        out_specs=pl.BlockSpec((r, o), lambda j: (j, 0)),
        out_shape=jax.ShapeDtypeStruct((bn_, o), jnp.float32),
    )(xg3, xt2, wt, gv, bv)


def _edge_conv(xt, xcm, w, gv, bv, o):
    b, n, c = xt.shape
    bn_ = b * n
    idx = _knn(xt, xcm)
    idx_nm = jnp.transpose(idx.reshape(bn_, KNN)).reshape(-1)
    cw = max(((c + 127) // 128) * 128, 128)
    x2 = xt.reshape(bn_, c)
    xpad = x2 if cw == c else jnp.pad(x2, ((0, 0), (0, cw - c)))
    xg = _sc_gather(xpad, idx_nm)
    row = lambda v: v.reshape(1, -1)
    out = _edge(xg.reshape(KNN, bn_, cw), x2, w.T, row(gv), row(bv), o)
    xt2 = out.reshape(b, n, o)
    return xt2, jnp.transpose(xt2, (0, 2, 1))


def _head_body(x1_ref, x2_ref, x3_ref, x4_ref, w5t_ref, g5_ref, b5_ref,
               fwt_ref, fb_ref, oga_ref, oba_ref, swt_ref, sb_ref,
               f0_ref, f1_ref, f2_ref, f3_ref, sem_ref):
    xc = jnp.concatenate(
        [x1_ref[0], x2_ref[0], x3_ref[0], x4_ref[0]], axis=1)
    h = jnp.dot(_bf(xc), _bf(w5t_ref[...]), preferred_element_type=jnp.float32)
    x5 = _lrelu(_bn(h, g5_ref[...], b5_ref[...]))
    f = jnp.dot(_bf(x5), _bf(fwt_ref[...]),
                preferred_element_type=jnp.float32) + fb_ref[...]
    fb_all = _bn(f, oga_ref[...], oba_ref[...])
    f0_ref[0] = fb_all[:, 0:256]
    f1_ref[0] = fb_all[:, 256:512]
    f2_ref[0] = fb_all[:, 512:768]
    f3_ref[0] = fb_all[:, 768:896]
    sem_ref[0] = jnp.dot(_bf(fb_all[:, 768:896]), _bf(swt_ref[...]),
                         preferred_element_type=jnp.float32) + sb_ref[...]


def _head(x1, x2, x3, x4, w5t, g5, b5, fwt, fb, oga, oba, swt, sb, r=512):
    b, n, _ = x1.shape

    def full(shp):
        return pl.BlockSpec(shp, lambda i, j: tuple(0 for _ in shp))

    return pl.pallas_call(
        _head_body,
        grid=(b, n // r),
        in_specs=[
            pl.BlockSpec((1, r, 64), lambda i, j: (i, j, 0)),
            pl.BlockSpec((1, r, 64), lambda i, j: (i, j, 0)),
            pl.BlockSpec((1, r, 128), lambda i, j: (i, j, 0)),
            pl.BlockSpec((1, r, 256), lambda i, j: (i, j, 0)),
            full((512, 512)), full((1, 512)), full((1, 512)),
            full((512, 896)), full((1, 896)), full((1, 896)), full((1, 896)),
            full((128, 20)), full((1, 20)),
        ],
        out_specs=[
            pl.BlockSpec((1, r, 256), lambda i, j: (i, j, 0)),
            pl.BlockSpec((1, r, 256), lambda i, j: (i, j, 0)),
            pl.BlockSpec((1, r, 256), lambda i, j: (i, j, 0)),
            pl.BlockSpec((1, r, 128), lambda i, j: (i, j, 0)),
            pl.BlockSpec((1, r, 20), lambda i, j: (i, j, 0)),
        ],
        out_shape=[
            jax.ShapeDtypeStruct((b, n, 256), jnp.float32),
            jax.ShapeDtypeStruct((b, n, 256), jnp.float32),
            jax.ShapeDtypeStruct((b, n, 256), jnp.float32),
            jax.ShapeDtypeStruct((b, n, 128), jnp.float32),
            jax.ShapeDtypeStruct((b, n, 20), jnp.float32),
        ],
    )(x1, x2, x3, x4, w5t, g5, b5, fwt, fb, oga, oba, swt, sb)


def kernel(x, W1, g1, b1, W2, g2, b2, W3, g3, b3, W4, g4, b4, W5, g5, b5,
           Fw0, Fb0, og0, ob0, Fw1, Fb1, og1, ob1, Fw2, Fb2, og2, ob2,
           Fw3, Fb3, og3, ob3, SW, Sb):
    b, _, n = x.shape
    xcm = jnp.concatenate([x, jnp.zeros((b, 5, n), jnp.float32)], axis=1)
    xt = jnp.transpose(xcm, (0, 2, 1))
    z5 = jnp.zeros((64, 5), jnp.float32)
    w1p = jnp.concatenate(
        [jnp.concatenate([W1[:, :3], z5], axis=1),
         jnp.concatenate([W1[:, 3:], z5], axis=1)], axis=1)

    x1t, x1c = _edge_conv(xt, xcm, w1p, g1, b1, 64)
    x2t, x2c = _edge_conv(x1t, x1c, W2, g2, b2, 64)
    x3t, x3c = _edge_conv(x2t, x2c, W3, g3, b3, 128)
    x4t, _ = _edge_conv(x3t, x3c, W4, g4, b4, 256)

    row = lambda v: v.reshape(1, -1)
    fwt = jnp.concatenate([Fw0, Fw1, Fw2, Fw3], axis=0).T
    fb = row(jnp.concatenate([Fb0, Fb1, Fb2, Fb3]))
    oga = row(jnp.concatenate([og0, og1, og2, og3]))
    oba = row(jnp.concatenate([ob0, ob1, ob2, ob3]))
    f0, f1, f2, f3, sem = _head(
        x1t, x2t, x3t, x4t, W5.T, row(g5), row(b5),
        fwt, fb, oga, oba, SW.T, row(Sb))
    tr = lambda v: jnp.transpose(v, (0, 2, 1))
    return (tr(f0), tr(f1), tr(f2), tr(f3), sem)
